# baseline (speedup 1.0000x reference)
import jax
import jax.numpy as jnp
from jax import lax
from jax.experimental import pallas as pl
from jax.experimental.pallas import tpu as pltpu
from jax.experimental.pallas import tpu_sc as plsc

B = 16384
H = 50
D = 64
DP = 128
NC, NS = 2, 16
NW = NC * NS
BPW = B // NW
G = 16
CHUNKS = BPW // G
GI = G * H
DMA_ROWS = 80
NDMA = GI // DMA_ROWS
NV = D // 16


def _compact_body(idx2d_hbm, idx1d_hbm, stage_v, list_v):
    wid = lax.axis_index("s") * NC + lax.axis_index("c")
    bag_base = wid * BPW

    def chunk(c, carry):
        row0 = bag_base + c * G
        pltpu.sync_copy(idx2d_hbm.at[pl.ds(row0, G)], stage_v)
        for r in range(G):
            for o in (0, 16, 32, H - 16):
                list_v[pl.ds(r * H + o, 16)] = stage_v[r, pl.ds(o, 16)]
        pltpu.sync_copy(list_v, idx1d_hbm.at[pl.ds(row0 * H, GI)])
        return carry

    lax.fori_loop(0, CHUNKS, chunk, 0)


_compact_call = pl.kernel(
    _compact_body,
    out_type=jax.ShapeDtypeStruct((B * H,), jnp.int32),
    mesh=plsc.VectorSubcoreMesh(
        core_axis_name="c", subcore_axis_name="s", num_cores=NC, num_subcores=NS
    ),
    scratch_types=[
        pltpu.VMEM((G, H), jnp.int32),
        pltpu.VMEM((GI,), jnp.int32),
    ],
    compiler_params=pltpu.CompilerParams(use_tc_tiling_on_sc=True),
)


def _body(weight_hbm, idx_hbm, out_hbm, idx_v, rows_v, out_v, sem0, sem1,
          osem0, osem1):
    wid = lax.axis_index("s") * NC + lax.axis_index("c")
    idx_base = wid * (BPW * H)
    out_base = wid * BPW
    sems = (sem0, sem1)
    osems = (osem0, osem1)

    def stage_fire(c, buf):
        pltpu.sync_copy(
            idx_hbm.at[pl.ds(idx_base + c * GI, GI)], idx_v.at[buf]
        )
        for j in range(NDMA):
            pltpu.async_copy(
                weight_hbm.at[idx_v.at[buf, pl.ds(j * DMA_ROWS, DMA_ROWS)]],
                rows_v.at[buf, pl.ds(j * DMA_ROWS, DMA_ROWS)],
                sems[buf],
            )

    def drain(buf):
        for j in range(NDMA):
            pltpu.make_async_copy(
                weight_hbm.at[idx_v.at[buf, pl.ds(j * DMA_ROWS, DMA_ROWS)]],
                rows_v.at[buf, pl.ds(j * DMA_ROWS, DMA_ROWS)],
                sems[buf],
            ).wait()

    def fire_out(c, buf):
        pltpu.async_copy(
            out_v.at[buf], out_hbm.at[pl.ds(out_base + c * G, G)], osems[buf]
        )

    def drain_out(buf):
        pltpu.make_async_copy(
            out_v.at[buf], out_hbm.at[pl.ds(out_base, G)], osems[buf]
        ).wait()

    def compute(c, buf):
        def bag(b, carry):
            rb = b * H
            acc = [rows_v[buf, rb, pl.ds(v * 16, 16)] for v in range(NV)]
            for l in range(1, H):
                for v in range(NV):
                    acc[v] = acc[v] + rows_v[buf, rb + l, pl.ds(v * 16, 16)]
            for v in range(NV):
                out_v[buf, b, pl.ds(v * 16, 16)] = acc[v] * (1.0 / H)
            return carry

        lax.fori_loop(0, G, bag, 0)

    zpad = jnp.zeros((16,), jnp.float32)
    for zb in range(2):
        def zrow(b, carry, _zb=zb):
            for v in range(NV, DP // 16):
                out_v[_zb, b, pl.ds(v * 16, 16)] = zpad
            return carry

        lax.fori_loop(0, G, zrow, 0)

    stage_fire(0, 0)

    def body(t, carry):
        c0 = 2 * t
        c1 = 2 * t + 1
        stage_fire(c1, 1)
        drain(0)

        @pl.when(t >= 1)
        def _():
            drain_out(0)

        compute(c0, 0)
        fire_out(c0, 0)

        @pl.when(c0 + 2 < CHUNKS)
        def _():
            stage_fire(c0 + 2, 0)

        drain(1)

        @pl.when(t >= 1)
        def _():
            drain_out(1)

        compute(c1, 1)
        fire_out(c1, 1)
        return carry

    lax.fori_loop(0, CHUNKS // 2, body, 0)
    drain_out(0)
    drain_out(1)


_sc_call = pl.kernel(
    _body,
    out_type=jax.ShapeDtypeStruct((B, DP), jnp.float32),
    mesh=plsc.VectorSubcoreMesh(
        core_axis_name="c", subcore_axis_name="s", num_cores=NC, num_subcores=NS
    ),
    scratch_types=[
        pltpu.VMEM((2, GI), jnp.int32),
        pltpu.VMEM((2, GI, D), jnp.float32),
        pltpu.VMEM((2, G, DP), jnp.float32),
        pltpu.SemaphoreType.DMA,
        pltpu.SemaphoreType.DMA,
        pltpu.SemaphoreType.DMA,
        pltpu.SemaphoreType.DMA,
    ],
    compiler_params=pltpu.CompilerParams(use_tc_tiling_on_sc=False),
)


def kernel(input, weight):
    idx = _compact_call(input.astype(jnp.int32))
    return _sc_call(weight, idx)[:, :D]
